# baseline (speedup 1.0000x reference)
import functools

import jax
import jax.numpy as jnp
from jax import lax
from jax.experimental import pallas as pl
from jax.experimental.pallas import tpu as pltpu
from jax.experimental.pallas import tpu_sc as plsc

N = 10000
D = 128
E = 320000

NCORE = 2
NSUB = 16
NW = NCORE * NSUB
CHUNK = 128
NH = 2
K = 80
KH = K // NH
EPW = K * CHUNK
NACC = NSUB * 640
ROWS_PER_TILE = 640


def _sc_scatter_body(vals_hbm, src_hbm, dst_hbm, zeros_hbm, out_hbm,
                     src_v, dst_v, rows_v, acc_sh, gsem, ssem):
    c = lax.axis_index("c")
    s = lax.axis_index("s")
    r0 = s * ROWS_PER_TILE
    pltpu.sync_copy(zeros_hbm.at[pl.ds(r0, ROWS_PER_TILE)],
                    acc_sh.at[pl.ds(r0, ROWS_PER_TILE)])
    plsc.subcore_barrier()

    for h in range(NH):
        pltpu.sync_copy(src_hbm.at[c, s, h], src_v)
        pltpu.sync_copy(dst_hbm.at[c, s, h], dst_v)
        @pl.loop(0, KH, step=2)
        def _(j0):
            d0 = pltpu.async_copy(vals_hbm.at[src_v.at[j0]], rows_v.at[0],
                                  gsem.at[0])
            d1 = pltpu.async_copy(vals_hbm.at[src_v.at[j0 + 1]],
                                  rows_v.at[1], gsem.at[1])
            d0.wait()
            d1.wait()
            e0 = pltpu.async_copy(rows_v.at[0], acc_sh.at[dst_v.at[j0]],
                                  ssem.at[0], add=True)
            e1 = pltpu.async_copy(rows_v.at[1], acc_sh.at[dst_v.at[j0 + 1]],
                                  ssem.at[1], add=True)
            e0.wait()
            e1.wait()

    plsc.subcore_barrier()
    pltpu.sync_copy(acc_sh.at[pl.ds(r0, ROWS_PER_TILE)],
                    out_hbm.at[c, pl.ds(r0, ROWS_PER_TILE)])


@functools.cache
def _get_sc_scatter():
    return pl.kernel(
        _sc_scatter_body,
        out_type=jax.ShapeDtypeStruct((NCORE, NACC, D), jnp.float32),
        mesh=plsc.VectorSubcoreMesh(core_axis_name="c", subcore_axis_name="s"),
        scratch_types=[
            pltpu.VMEM((KH, CHUNK), jnp.int32),
            pltpu.VMEM((KH, CHUNK), jnp.int32),
            pltpu.VMEM((2, CHUNK, D), jnp.float32),
            pltpu.VMEM_SHARED((NACC, D), jnp.float32),
            pltpu.SemaphoreType.DMA((2,)),
            pltpu.SemaphoreType.DMA((2,)),
        ],
    )


def _dotT(a, w):
    return lax.dot_general(a, w, (((1,), (1,)), ((), ())),
                           preferred_element_type=jnp.float32)


def _tc_hneigh_body(x_ref, w_ref, b_ref, o_ref):
    o_ref[...] = _dotT(x_ref[...], w_ref[...]) + b_ref[...]


def _tc_dense_body(xm_ref, wsn_ref, bsn_ref, wsl_ref, bsl_ref,
                   wco_ref, bco_ref, o_ref):
    xm = xm_ref[...]
    hm = _dotT(xm, wsn_ref[...]) + bsn_ref[...]
    zrow = jnp.zeros((1, D), jnp.float32)
    up = jnp.concatenate([hm[1:], zrow], axis=0)
    dn = jnp.concatenate([zrow, hm[:-1]], axis=0)
    row = lax.broadcasted_iota(jnp.int32, (N, 1), 0)
    cnt = jnp.where((row == 0) | (row == N - 1), 1.0, 2.0)
    sarr = (xm + up + dn) / cnt
    wsl = wsl_ref[...]
    h_seq = _dotT(xm, wsl[:, :D]) + _dotT(sarr, wsl[:, D:]) + bsl_ref[...]
    wco = wco_ref[...]
    o_ref[...] = (_dotT(xm, wco[:, D:2 * D]) + _dotT(h_seq, wco[:, 2 * D:])
                  + bco_ref[...])


def _tc_hfinal_body(p_ref, hpre_ref, wco_ref, scale_ref, bias_ref, o_ref):
    hs = p_ref[0, :N, :] + p_ref[1, :N, :]
    h = _dotT(hs, wco_ref[...][:, :D]) + hpre_ref[...]
    o_ref[...] = h * scale_ref[...] + bias_ref[...]


def _tc_sum_body(q_ref, o_ref):
    o_ref[...] = q_ref[0, :N, :] + q_ref[1, :N, :]


def _tc_call(body, out_shape, *args):
    return pl.pallas_call(body, out_shape=out_shape)(*args)


def kernel(x_metrical, x, edge_index, W_neigh, b_neigh, W_conv_out,
           b_conv_out, W_seq_neigh, b_seq_neigh, W_seq_lin, b_seq_lin,
           bn_weight, bn_bias):
    f32 = jnp.float32
    src = edge_index[0]
    dst = edge_index[1]
    pad = NW * EPW - E
    srcp = jnp.concatenate([src, jnp.zeros((pad,), jnp.int32)])
    dstp = jnp.concatenate([dst, jnp.full((pad,), N, jnp.int32)])
    src4 = srcp.reshape(NCORE, NSUB, NH, KH, CHUNK)
    dst4 = dstp.reshape(NCORE, NSUB, NH, KH, CHUNK)
    zeros_acc = jnp.zeros((NACC, D), f32)

    out_nd = jax.ShapeDtypeStruct((N, D), f32)

    h_neigh = _tc_call(_tc_hneigh_body, out_nd, x, W_neigh,
                       b_neigh.reshape(1, D))

    h_pre = _tc_call(_tc_dense_body, out_nd, x_metrical, W_seq_neigh,
                     b_seq_neigh.reshape(1, D), W_seq_lin,
                     b_seq_lin.reshape(1, D), W_conv_out,
                     b_conv_out.reshape(1, D))

    sc_scatter = _get_sc_scatter()
    p = sc_scatter(h_neigh, src4, dst4, zeros_acc)

    scale = (bn_weight * (1.0 / jnp.sqrt(1.0 + 1e-5))).reshape(1, D)
    h = _tc_call(_tc_hfinal_body, out_nd, p, h_pre, W_conv_out, scale,
                 bn_bias.reshape(1, D))

    q = sc_scatter(h, src4, dst4, zeros_acc)
    out = _tc_call(_tc_sum_body, out_nd, q)
    return (out, h)


# speedup vs baseline: 2.5064x; 2.5064x over previous
import functools

import jax
import jax.numpy as jnp
from jax import lax
from jax.experimental import pallas as pl
from jax.experimental.pallas import tpu as pltpu
from jax.experimental.pallas import tpu_sc as plsc

N = 10000
D = 128
E = 320000

NCORE = 2
NSUB = 16
NW = NCORE * NSUB
CHUNK = 128
K = 79
EPW = K * CHUNK
NACC = NSUB * 640
ROWS_PER_TILE = 640


def _sc_scatter_body(vals_hbm, src_hbm, dst_hbm, zeros_hbm, out_hbm,
                     src_v, dst_v, rows_v, acc_sh, gsem):
    c = lax.axis_index("c")
    s = lax.axis_index("s")
    r0 = s * ROWS_PER_TILE
    pltpu.sync_copy(zeros_hbm.at[pl.ds(r0, ROWS_PER_TILE)],
                    acc_sh.at[pl.ds(r0, ROWS_PER_TILE)])
    plsc.subcore_barrier()

    pltpu.sync_copy(src_hbm.at[c, s], src_v)
    pltpu.sync_copy(dst_hbm.at[c, s], dst_v)
    plsc.subcore_barrier()

    @pl.loop(0, K)
    def _(j):
        pltpu.async_copy(vals_hbm.at[src_v.at[j]], rows_v.at[0],
                         gsem.at[0]).wait()
        pltpu.sync_copy(rows_v.at[0], acc_sh.at[dst_v.at[j]], add=True)

    plsc.subcore_barrier()
    pltpu.sync_copy(acc_sh.at[pl.ds(r0, ROWS_PER_TILE)],
                    out_hbm.at[c, pl.ds(r0, ROWS_PER_TILE)])


@functools.cache
def _get_sc_scatter():
    return pl.kernel(
        _sc_scatter_body,
        out_type=jax.ShapeDtypeStruct((NCORE, NACC, D), jnp.float32),
        mesh=plsc.VectorSubcoreMesh(core_axis_name="c", subcore_axis_name="s"),
        scratch_types=[
            pltpu.VMEM((K, CHUNK), jnp.int32),
            pltpu.VMEM((K, CHUNK), jnp.int32),
            pltpu.VMEM((1, CHUNK, D), jnp.float32),
            pltpu.VMEM_SHARED((NACC, D), jnp.float32),
            pltpu.SemaphoreType.DMA((2,)),
        ],
    )


def _dotT(a, w):
    return lax.dot_general(a, w, (((1,), (1,)), ((), ())),
                           preferred_element_type=jnp.float32)


def _tc_hneigh_body(x_ref, w_ref, b_ref, o_ref):
    o_ref[...] = _dotT(x_ref[...], w_ref[...]) + b_ref[...]


def _tc_dense_body(xm_ref, wsn_ref, bsn_ref, wsl_ref, bsl_ref,
                   wco_ref, bco_ref, o_ref):
    xm = xm_ref[...]
    hm = _dotT(xm, wsn_ref[...]) + bsn_ref[...]
    zrow = jnp.zeros((1, D), jnp.float32)
    up = jnp.concatenate([hm[1:], zrow], axis=0)
    dn = jnp.concatenate([zrow, hm[:-1]], axis=0)
    row = lax.broadcasted_iota(jnp.int32, (N, 1), 0)
    cnt = jnp.where((row == 0) | (row == N - 1), 1.0, 2.0)
    sarr = (xm + up + dn) / cnt
    wsl = wsl_ref[...]
    h_seq = _dotT(xm, wsl[:, :D]) + _dotT(sarr, wsl[:, D:]) + bsl_ref[...]
    wco = wco_ref[...]
    o_ref[...] = (_dotT(xm, wco[:, D:2 * D]) + _dotT(h_seq, wco[:, 2 * D:])
                  + bco_ref[...])


def _tc_hfinal_body(p_ref, hpre_ref, wco_ref, scale_ref, bias_ref, o_ref):
    hs = p_ref[0, :N, :] + p_ref[1, :N, :]
    h = _dotT(hs, wco_ref[...][:, :D]) + hpre_ref[...]
    o_ref[...] = h * scale_ref[...] + bias_ref[...]


def _tc_sum_body(q_ref, o_ref):
    o_ref[...] = q_ref[0, :N, :] + q_ref[1, :N, :]


def _tc_call(body, out_shape, *args):
    return pl.pallas_call(body, out_shape=out_shape)(*args)


def kernel(x_metrical, x, edge_index, W_neigh, b_neigh, W_conv_out,
           b_conv_out, W_seq_neigh, b_seq_neigh, W_seq_lin, b_seq_lin,
           bn_weight, bn_bias):
    f32 = jnp.float32
    src = edge_index[0]
    dst = edge_index[1]
    pad = NW * EPW - E
    pad_i = jnp.arange(pad, dtype=jnp.int32)
    srcp = jnp.concatenate([src, pad_i % 8])
    dstp = jnp.concatenate([dst, N + pad_i % (NACC - N)])
    src4 = srcp.reshape(NCORE, NSUB, K, CHUNK)
    dst4 = dstp.reshape(NCORE, NSUB, K, CHUNK)
    zeros_acc = jnp.zeros((NACC, D), f32)

    out_nd = jax.ShapeDtypeStruct((N, D), f32)

    h_neigh = _tc_call(_tc_hneigh_body, out_nd, x, W_neigh,
                       b_neigh.reshape(1, D))

    h_pre = _tc_call(_tc_dense_body, out_nd, x_metrical, W_seq_neigh,
                     b_seq_neigh.reshape(1, D), W_seq_lin,
                     b_seq_lin.reshape(1, D), W_conv_out,
                     b_conv_out.reshape(1, D))

    sc_scatter = _get_sc_scatter()
    p = sc_scatter(h_neigh, src4, dst4, zeros_acc)

    scale = (bn_weight * (1.0 / jnp.sqrt(1.0 + 1e-5))).reshape(1, D)
    h = _tc_call(_tc_hfinal_body, out_nd, p, h_pre, W_conv_out, scale,
                 bn_bias.reshape(1, D))

    q = sc_scatter(h, src4, dst4, zeros_acc)
    out = _tc_call(_tc_sum_body, out_nd, q)
    return (out, h)


# speedup vs baseline: 3.2173x; 1.2836x over previous
import functools

import jax
import jax.numpy as jnp
from jax import lax
from jax.experimental import pallas as pl
from jax.experimental.pallas import tpu as pltpu
from jax.experimental.pallas import tpu_sc as plsc

N = 10000
D = 128
E = 320000

NCORE = 2
NSUB = 16
NW = NCORE * NSUB
CHUNK = 128
NH = 2
K = 80
KH = K // NH
EPW = K * CHUNK
NACC = NSUB * 640
ROWS_PER_TILE = 640


def _sc_scatter_body(vals_hbm, src_hbm, dst_hbm, zeros_hbm, out_hbm,
                     src_v, dst_v, rows_v, acc_sh, gsem):
    c = lax.axis_index("c")
    s = lax.axis_index("s")
    r0 = s * ROWS_PER_TILE
    pltpu.sync_copy(zeros_hbm.at[pl.ds(r0, ROWS_PER_TILE)],
                    acc_sh.at[pl.ds(r0, ROWS_PER_TILE)])
    plsc.subcore_barrier()

    plsc.subcore_barrier()

    for h in range(NH):
        pltpu.sync_copy(src_hbm.at[c, s, h], src_v)
        pltpu.sync_copy(dst_hbm.at[c, s, h], dst_v)
        pltpu.async_copy(vals_hbm.at[src_v.at[0]], rows_v.at[0], gsem.at[0])

        @pl.loop(0, KH)
        def _(j):
            b = lax.rem(j, 2)
            nb = 1 - b
            jn = lax.rem(j + 1, KH)
            pltpu.async_copy(vals_hbm.at[src_v.at[jn]], rows_v.at[nb],
                             gsem.at[nb])
            pltpu.make_async_copy(vals_hbm.at[src_v.at[j]], rows_v.at[b],
                                  gsem.at[b]).wait()
            pltpu.sync_copy(rows_v.at[b], acc_sh.at[dst_v.at[j]], add=True)

        pltpu.make_async_copy(vals_hbm.at[src_v.at[0]], rows_v.at[0],
                              gsem.at[0]).wait()

    plsc.subcore_barrier()
    pltpu.sync_copy(acc_sh.at[pl.ds(r0, ROWS_PER_TILE)],
                    out_hbm.at[c, pl.ds(r0, ROWS_PER_TILE)])


@functools.cache
def _get_sc_scatter():
    return pl.kernel(
        _sc_scatter_body,
        out_type=jax.ShapeDtypeStruct((NCORE, NACC, D), jnp.float32),
        mesh=plsc.VectorSubcoreMesh(core_axis_name="c", subcore_axis_name="s"),
        scratch_types=[
            pltpu.VMEM((KH, CHUNK), jnp.int32),
            pltpu.VMEM((KH, CHUNK), jnp.int32),
            pltpu.VMEM((2, CHUNK, D), jnp.float32),
            pltpu.VMEM_SHARED((NACC, D), jnp.float32),
            pltpu.SemaphoreType.DMA((2,)),
        ],
    )


def _dotT(a, w):
    return lax.dot_general(a, w, (((1,), (1,)), ((), ())),
                           preferred_element_type=jnp.float32)


def _tc_hneigh_body(x_ref, w_ref, b_ref, o_ref):
    o_ref[...] = _dotT(x_ref[...], w_ref[...]) + b_ref[...]


def _tc_dense_body(xm_ref, wsn_ref, bsn_ref, wsl_ref, bsl_ref,
                   wco_ref, bco_ref, o_ref):
    xm = xm_ref[...]
    hm = _dotT(xm, wsn_ref[...]) + bsn_ref[...]
    zrow = jnp.zeros((1, D), jnp.float32)
    up = jnp.concatenate([hm[1:], zrow], axis=0)
    dn = jnp.concatenate([zrow, hm[:-1]], axis=0)
    row = lax.broadcasted_iota(jnp.int32, (N, 1), 0)
    cnt = jnp.where((row == 0) | (row == N - 1), 1.0, 2.0)
    sarr = (xm + up + dn) / cnt
    wsl = wsl_ref[...]
    h_seq = _dotT(xm, wsl[:, :D]) + _dotT(sarr, wsl[:, D:]) + bsl_ref[...]
    wco = wco_ref[...]
    o_ref[...] = (_dotT(xm, wco[:, D:2 * D]) + _dotT(h_seq, wco[:, 2 * D:])
                  + bco_ref[...])


def _tc_hfinal_body(p_ref, hpre_ref, wco_ref, scale_ref, bias_ref, o_ref):
    hs = p_ref[0, :N, :] + p_ref[1, :N, :]
    h = _dotT(hs, wco_ref[...][:, :D]) + hpre_ref[...]
    o_ref[...] = h * scale_ref[...] + bias_ref[...]


def _tc_sum_body(q_ref, o_ref):
    o_ref[...] = q_ref[0, :N, :] + q_ref[1, :N, :]


def _tc_call(body, out_shape, *args):
    return pl.pallas_call(body, out_shape=out_shape)(*args)


def kernel(x_metrical, x, edge_index, W_neigh, b_neigh, W_conv_out,
           b_conv_out, W_seq_neigh, b_seq_neigh, W_seq_lin, b_seq_lin,
           bn_weight, bn_bias):
    f32 = jnp.float32
    src = edge_index[0]
    dst = edge_index[1]
    pad = NW * EPW - E
    pad_i = jnp.arange(pad, dtype=jnp.int32)
    srcp = jnp.concatenate([src, pad_i % 8])
    dstp = jnp.concatenate([dst, N + pad_i % (NACC - N)])
    src4 = srcp.reshape(NCORE, NSUB, NH, KH, CHUNK)
    dst4 = dstp.reshape(NCORE, NSUB, NH, KH, CHUNK)
    zeros_acc = jnp.zeros((NACC, D), f32)

    out_nd = jax.ShapeDtypeStruct((N, D), f32)

    h_neigh = _tc_call(_tc_hneigh_body, out_nd, x, W_neigh,
                       b_neigh.reshape(1, D))

    h_pre = _tc_call(_tc_dense_body, out_nd, x_metrical, W_seq_neigh,
                     b_seq_neigh.reshape(1, D), W_seq_lin,
                     b_seq_lin.reshape(1, D), W_conv_out,
                     b_conv_out.reshape(1, D))

    sc_scatter = _get_sc_scatter()
    p = sc_scatter(h_neigh, src4, dst4, zeros_acc)

    scale = (bn_weight * (1.0 / jnp.sqrt(1.0 + 1e-5))).reshape(1, D)
    h = _tc_call(_tc_hfinal_body, out_nd, p, h_pre, W_conv_out, scale,
                 bn_bias.reshape(1, D))

    q = sc_scatter(h, src4, dst4, zeros_acc)
    out = _tc_call(_tc_sum_body, out_nd, q)
    return (out, h)
